# Initial kernel scaffold; baseline (speedup 1.0000x reference)
import jax
import jax.numpy as jnp
from jax.experimental import pallas as pl


def kernel(u_table, v_table, freq_emb, time, pos_u, pos_v, neg_v):
    raise NotImplementedError("write your pallas kernel here")


# speedup vs baseline: 2.7476x; 2.7476x over previous
import functools

import jax
import jax.numpy as jnp
from jax import lax
from jax.experimental import pallas as pl
from jax.experimental.pallas import tpu as pltpu
from jax.experimental.pallas import tpu_sc as plsc

VOCAB = 100000
D = 128
B = 16384
NEG = 5

NC = 2
NS = 16
NW = NC * NS
CHUNK = 128

U_PER_W = B // NW
N_PER_W = B * NEG // NW


def _sc_gather_body(u_hbm, v_hbm, pu_hbm, pv_hbm, nf_hbm,
                    ug_hbm, vg_hbm, ng_hbm, idx_v, rows_v, sem):
  c = lax.axis_index("c")
  s = lax.axis_index("s")
  wid = s * NC + c

  def gather_stream(idx_src, n_rows, table, out_hbm, base):
    for j in range(n_rows // CHUNK):
      off = base + j * CHUNK
      pltpu.sync_copy(idx_src.at[pl.ds(off, CHUNK)], idx_v)
      pltpu.async_copy(table.at[idx_v], rows_v, sem).wait()
      pltpu.sync_copy(rows_v, out_hbm.at[pl.ds(off, CHUNK)])

  gather_stream(pu_hbm, U_PER_W, u_hbm, ug_hbm, wid * U_PER_W)
  gather_stream(pv_hbm, U_PER_W, v_hbm, vg_hbm, wid * U_PER_W)
  gather_stream(nf_hbm, N_PER_W, v_hbm, ng_hbm, wid * N_PER_W)


def _sc_gather(u_table, v_table, pos_u, pos_v, neg_flat):
  mesh = plsc.VectorSubcoreMesh(core_axis_name="c", subcore_axis_name="s")
  out_type = [
      jax.ShapeDtypeStruct((B, D), jnp.float32),
      jax.ShapeDtypeStruct((B, D), jnp.float32),
      jax.ShapeDtypeStruct((B * NEG, D), jnp.float32),
  ]
  k = pl.kernel(
      _sc_gather_body,
      out_type=out_type,
      mesh=mesh,
      scratch_types=[
          pltpu.VMEM((CHUNK,), jnp.int32),
          pltpu.VMEM((CHUNK, D), jnp.float32),
          pltpu.SemaphoreType.DMA,
      ],
  )
  return k(u_table, v_table, pos_u, pos_v, neg_flat)


CB = 512
NBLK = B // CB


def _tc_loss_body(t_ref, f_ref, ug_ref, vg_ref, n0, n1, n2, n3, n4, o_ref):
  i = pl.program_id(0)
  te = jnp.sin(t_ref[...] * f_ref[...])
  eu = ug_ref[...] + te
  s = jnp.clip(jnp.sum(eu * vg_ref[...], axis=-1), -10.0, 10.0)
  acc = jnp.sum(jnp.log1p(jnp.exp(-s)))
  for nref in (n0, n1, n2, n3, n4):
    ns = jnp.clip(jnp.sum(nref[...] * eu, axis=-1), -10.0, 10.0)
    acc = acc + jnp.sum(jnp.log1p(jnp.exp(ns)))

  @pl.when(i == 0)
  def _():
    o_ref[0, 0] = 0.0

  o_ref[0, 0] += acc


def _tc_loss(time, freq_emb, ug, vg, ng):
  t2 = time.reshape(B, 1)
  f2 = freq_emb.reshape(1, D)
  in_specs = [
      pl.BlockSpec((CB, 1), lambda i: (i, 0)),
      pl.BlockSpec((1, D), lambda i: (0, 0)),
      pl.BlockSpec((CB, D), lambda i: (i, 0)),
      pl.BlockSpec((CB, D), lambda i: (i, 0)),
  ] + [
      pl.BlockSpec((CB, D), lambda i, k=k: (k * NBLK + i, 0))
      for k in range(NEG)
  ]
  out = pl.pallas_call(
      _tc_loss_body,
      grid=(NBLK,),
      in_specs=in_specs,
      out_specs=pl.BlockSpec((1, 1), lambda i: (0, 0),
                             memory_space=pltpu.SMEM),
      out_shape=jax.ShapeDtypeStruct((1, 1), jnp.float32),
  )(t2, f2, ug, vg, ng, ng, ng, ng, ng)
  return out


def kernel(u_table, v_table, freq_emb, time, pos_u, pos_v, neg_v):
  pu = pos_u.astype(jnp.int32)
  pv = pos_v.astype(jnp.int32)
  nf = neg_v.T.reshape(-1).astype(jnp.int32)
  ug, vg, ng = _sc_gather(u_table, v_table, pu, pv, nf)
  acc = _tc_loss(time, freq_emb, ug, vg, ng)
  return acc[0, 0] / B


# speedup vs baseline: 3.4180x; 1.2440x over previous
import functools

import jax
import jax.numpy as jnp
from jax import lax
from jax.experimental import pallas as pl
from jax.experimental.pallas import tpu as pltpu
from jax.experimental.pallas import tpu_sc as plsc

VOCAB = 100000
D = 128
B = 16384
NEG = 5

NC = 2
NS = 16
NW = NC * NS
CHUNK = 128

U_PER_W = B // NW
N_PER_W = B * NEG // NW


DEPTH = 6
N_CHUNKS = (2 * U_PER_W + N_PER_W) // CHUNK


def _sc_gather_body(u_hbm, v_hbm, pu_hbm, pv_hbm, nf_hbm,
                    ug_hbm, vg_hbm, ng_hbm,
                    idxu, idxv, idxn,
                    b0, b1, b2, b3, b4, b5,
                    g0, g1, g2, g3, g4, g5,
                    w0, w1, w2, w3, w4, w5):
  bufs = [b0, b1, b2, b3, b4, b5]
  gsem = [g0, g1, g2, g3, g4, g5]
  wsem = [w0, w1, w2, w3, w4, w5]
  c = lax.axis_index("c")
  s = lax.axis_index("s")
  wid = s * NC + c

  h0 = pltpu.async_copy(pu_hbm.at[pl.ds(wid * U_PER_W, U_PER_W)], idxu, wsem[0])
  h1 = pltpu.async_copy(pv_hbm.at[pl.ds(wid * U_PER_W, U_PER_W)], idxv, wsem[1])
  h2 = pltpu.async_copy(nf_hbm.at[pl.ds(wid * N_PER_W, N_PER_W)], idxn, wsem[2])
  h0.wait()
  h1.wait()
  h2.wait()

  chunks = []
  for j in range(U_PER_W // CHUNK):
    chunks.append((u_hbm, idxu, j * CHUNK, ug_hbm, wid * U_PER_W + j * CHUNK))
  for j in range(U_PER_W // CHUNK):
    chunks.append((v_hbm, idxv, j * CHUNK, vg_hbm, wid * U_PER_W + j * CHUNK))
  for j in range(N_PER_W // CHUNK):
    chunks.append((v_hbm, idxn, j * CHUNK, ng_hbm, wid * N_PER_W + j * CHUNK))

  gh = [None] * N_CHUNKS
  wh = [None] * N_CHUNKS

  def start_gather(t):
    tbl, iref, ioff, _, _ = chunks[t]
    b = t % DEPTH
    gh[t] = pltpu.async_copy(tbl.at[iref.at[pl.ds(ioff, CHUNK)]],
                             bufs[b], gsem[b])

  for t in range(DEPTH):
    start_gather(t)
  for t in range(N_CHUNKS):
    b = t % DEPTH
    gh[t].wait()
    _, _, _, out_hbm, ooff = chunks[t]
    wh[t] = pltpu.async_copy(bufs[b], out_hbm.at[pl.ds(ooff, CHUNK)], wsem[b])
    if t + DEPTH < N_CHUNKS:
      wh[t].wait()
      start_gather(t + DEPTH)
  for t in range(N_CHUNKS - DEPTH, N_CHUNKS):
    wh[t].wait()


def _sc_gather(u_table, v_table, pos_u, pos_v, neg_flat):
  mesh = plsc.VectorSubcoreMesh(core_axis_name="c", subcore_axis_name="s")
  out_type = [
      jax.ShapeDtypeStruct((B, D), jnp.float32),
      jax.ShapeDtypeStruct((B, D), jnp.float32),
      jax.ShapeDtypeStruct((B * NEG, D), jnp.float32),
  ]
  k = pl.kernel(
      _sc_gather_body,
      out_type=out_type,
      mesh=mesh,
      scratch_types=(
          [pltpu.VMEM((U_PER_W,), jnp.int32),
           pltpu.VMEM((U_PER_W,), jnp.int32),
           pltpu.VMEM((N_PER_W,), jnp.int32)]
          + [pltpu.VMEM((CHUNK, D), jnp.float32) for _ in range(DEPTH)]
          + [pltpu.SemaphoreType.DMA for _ in range(2 * DEPTH)]
      ),
  )
  return k(u_table, v_table, pos_u, pos_v, neg_flat)


CB = 512
NBLK = B // CB


def _tc_loss_body(t_ref, f_ref, ug_ref, vg_ref, n0, n1, n2, n3, n4, o_ref):
  i = pl.program_id(0)
  te = jnp.sin(t_ref[...] * f_ref[...])
  eu = ug_ref[...] + te
  s = jnp.clip(jnp.sum(eu * vg_ref[...], axis=-1), -10.0, 10.0)
  acc = jnp.sum(jnp.log1p(jnp.exp(-s)))
  for nref in (n0, n1, n2, n3, n4):
    ns = jnp.clip(jnp.sum(nref[...] * eu, axis=-1), -10.0, 10.0)
    acc = acc + jnp.sum(jnp.log1p(jnp.exp(ns)))

  @pl.when(i == 0)
  def _():
    o_ref[0, 0] = 0.0

  o_ref[0, 0] += acc


def _tc_loss(time, freq_emb, ug, vg, ng):
  t2 = time.reshape(B, 1)
  f2 = freq_emb.reshape(1, D)
  in_specs = [
      pl.BlockSpec((CB, 1), lambda i: (i, 0)),
      pl.BlockSpec((1, D), lambda i: (0, 0)),
      pl.BlockSpec((CB, D), lambda i: (i, 0)),
      pl.BlockSpec((CB, D), lambda i: (i, 0)),
  ] + [
      pl.BlockSpec((CB, D), lambda i, k=k: (k * NBLK + i, 0))
      for k in range(NEG)
  ]
  out = pl.pallas_call(
      _tc_loss_body,
      grid=(NBLK,),
      in_specs=in_specs,
      out_specs=pl.BlockSpec((1, 1), lambda i: (0, 0),
                             memory_space=pltpu.SMEM),
      out_shape=jax.ShapeDtypeStruct((1, 1), jnp.float32),
  )(t2, f2, ug, vg, ng, ng, ng, ng, ng)
  return out


def kernel(u_table, v_table, freq_emb, time, pos_u, pos_v, neg_v):
  pu = pos_u.astype(jnp.int32)
  pv = pos_v.astype(jnp.int32)
  nf = neg_v.T.reshape(-1).astype(jnp.int32)
  ug, vg, ng = _sc_gather(u_table, v_table, pu, pv, nf)
  acc = _tc_loss(time, freq_emb, ug, vg, ng)
  return acc[0, 0] / B
